# baseline (speedup 1.0000x reference)
import functools

import jax
import jax.numpy as jnp
from jax.experimental import pallas as pl
from jax.experimental.pallas import tpu as pltpu

_N_BLK = 2048


def _detection_kernel(rois_ref, ids_ref, gt_ref, out_ref):
    r = rois_ref[0]
    y1 = r[0:1, :]
    x1 = r[1:2, :]
    y2 = r[2:3, :]
    x2 = r[3:4, :]
    g = gt_ref[0]
    gy1 = g[:, 0:1]
    gx1 = g[:, 1:2]
    gy2 = g[:, 2:3]
    gx2 = g[:, 3:4]
    ids = ids_ref[0]

    iy1 = jnp.maximum(y1, gy1)
    ix1 = jnp.maximum(x1, gx1)
    iy2 = jnp.minimum(y2, gy2)
    ix2 = jnp.minimum(x2, gx2)
    inter = jnp.maximum(iy2 - iy1, 0.0) * jnp.maximum(ix2 - ix1, 0.0)
    a1 = (y2 - y1) * (x2 - x1)
    a2 = (gy2 - gy1) * (gx2 - gx1)
    union = a1 + a2 - inter
    iou = inter / jnp.maximum(union, 1e-8)

    gt_valid = (jnp.abs(gy1) > 0) | (jnp.abs(gx1) > 0) | \
               (jnp.abs(gy2) > 0) | (jnp.abs(gx2) > 0)
    nc_mask = gt_valid & (ids > 0)
    c_mask = gt_valid & (ids < 0)
    iou_nc = jnp.where(nc_mask, iou, -1.0)
    iou_c = jnp.where(c_mask, iou, -1.0)
    nc_max = jnp.max(iou_nc, axis=0, keepdims=True)
    c_max = jnp.max(iou_c, axis=0, keepdims=True)

    roi_valid = (jnp.abs(y1) > 0) | (jnp.abs(x1) > 0) | \
                (jnp.abs(y2) > 0) | (jnp.abs(x2) > 0)
    neg_one = jnp.float32(-1.0)
    nc_max = jnp.where(roi_valid, nc_max, neg_one)
    c_max = jnp.where(roi_valid, c_max, neg_one)
    pos = ((nc_max >= 0.5) & roi_valid).astype(jnp.float32)
    neg = ((nc_max < 0.5) & (c_max < 0.001) & roi_valid).astype(jnp.float32)
    out_ref[0] = jnp.concatenate([nc_max, c_max, pos, neg], axis=0)


@functools.partial(jax.jit, static_argnames=())
def kernel(rois, gt_ids, gt_boxes):
    B, N, _ = rois.shape
    G = gt_boxes.shape[1]
    n_pad = (N + _N_BLK - 1) // _N_BLK * _N_BLK
    rois_t = jnp.transpose(rois, (0, 2, 1))
    if n_pad != N:
        rois_t = jnp.pad(rois_t, ((0, 0), (0, 0), (0, n_pad - N)))
    ids3 = gt_ids.reshape(B, G, 1)
    nb = n_pad // _N_BLK
    out = pl.pallas_call(
        _detection_kernel,
        grid=(B, nb),
        in_specs=[
            pl.BlockSpec((1, 4, _N_BLK), lambda b, n: (b, 0, n)),
            pl.BlockSpec((1, G, 1), lambda b, n: (b, 0, 0)),
            pl.BlockSpec((1, G, 4), lambda b, n: (b, 0, 0)),
        ],
        out_specs=pl.BlockSpec((1, 4, _N_BLK), lambda b, n: (b, 0, n)),
        out_shape=jax.ShapeDtypeStruct((B, 4, n_pad), jnp.float32),
        compiler_params=pltpu.CompilerParams(
            dimension_semantics=("parallel", "parallel"),
        ),
    )(rois_t, ids3, gt_boxes)
    return out[:, :, :N]


# speedup vs baseline: 1.0840x; 1.0840x over previous
import functools

import jax
import jax.numpy as jnp
from jax.experimental import pallas as pl
from jax.experimental.pallas import tpu as pltpu

_N_BLK = 2048


def _detection_kernel(rois_ref, ids_ref, gt_ref, out_ref):
    r = rois_ref[0]
    y1 = r[0:1, :]
    x1 = r[1:2, :]
    y2 = r[2:3, :]
    x2 = r[3:4, :]
    g = gt_ref[0]
    gy1 = g[:, 0:1]
    gx1 = g[:, 1:2]
    gy2 = g[:, 2:3]
    gx2 = g[:, 3:4]
    ids = ids_ref[0]

    iy1 = jnp.maximum(y1, gy1)
    ix1 = jnp.maximum(x1, gx1)
    iy2 = jnp.minimum(y2, gy2)
    ix2 = jnp.minimum(x2, gx2)
    inter = jnp.maximum(iy2 - iy1, 0.0) * jnp.maximum(ix2 - ix1, 0.0)
    a1 = (y2 - y1) * (x2 - x1)
    a2 = (gy2 - gy1) * (gx2 - gx1)
    union = a1 + a2 - inter
    iou = inter / jnp.maximum(union, 1e-8)

    gt_valid = (jnp.abs(gy1) > 0) | (jnp.abs(gx1) > 0) | \
               (jnp.abs(gy2) > 0) | (jnp.abs(gx2) > 0)
    nc_mask = gt_valid & (ids > 0)
    c_mask = gt_valid & (ids < 0)
    iou_nc = jnp.where(nc_mask, iou, -1.0)
    iou_c = jnp.where(c_mask, iou, -1.0)
    nc_max = jnp.max(iou_nc, axis=0, keepdims=True)
    c_max = jnp.max(iou_c, axis=0, keepdims=True)

    roi_valid = (jnp.abs(y1) > 0) | (jnp.abs(x1) > 0) | \
                (jnp.abs(y2) > 0) | (jnp.abs(x2) > 0)
    neg_one = jnp.float32(-1.0)
    nc_max = jnp.where(roi_valid, nc_max, neg_one)
    c_max = jnp.where(roi_valid, c_max, neg_one)
    pos = ((nc_max >= 0.5) & roi_valid).astype(jnp.float32)
    neg = ((nc_max < 0.5) & (c_max < 0.001) & roi_valid).astype(jnp.float32)
    out_ref[0] = jnp.concatenate([nc_max, c_max, pos, neg], axis=0)


@functools.partial(jax.jit, static_argnames=())
def kernel(rois, gt_ids, gt_boxes):
    B, N, _ = rois.shape
    G = gt_boxes.shape[1]
    rois_t = jnp.transpose(rois, (0, 2, 1))
    ids3 = gt_ids.reshape(B, G, 1)
    nb = (N + _N_BLK - 1) // _N_BLK
    out = pl.pallas_call(
        _detection_kernel,
        grid=(B, nb),
        in_specs=[
            pl.BlockSpec((1, 4, _N_BLK), lambda b, n: (b, 0, n)),
            pl.BlockSpec((1, G, 1), lambda b, n: (b, 0, 0)),
            pl.BlockSpec((1, G, 4), lambda b, n: (b, 0, 0)),
        ],
        out_specs=pl.BlockSpec((1, 4, _N_BLK), lambda b, n: (b, 0, n)),
        out_shape=jax.ShapeDtypeStruct((B, 4, N), jnp.float32),
        compiler_params=pltpu.CompilerParams(
            dimension_semantics=("parallel", "parallel"),
        ),
    )(rois_t, ids3, gt_boxes)
    return out


# speedup vs baseline: 1.2599x; 1.1623x over previous
import jax
import jax.numpy as jnp
from jax.experimental import pallas as pl
from jax.experimental.pallas import tpu as pltpu


def _detection_kernel(rois_ref, ids_ref, gt_ref, out_ref, nc_ref, c_ref):
    r = rois_ref[0]
    y1 = r[0]
    x1 = r[1]
    y2 = r[2]
    x2 = r[3]
    a1 = (y2 - y1) * (x2 - x1)
    nc_ref[...] = jnp.full_like(nc_ref, -1.0)
    c_ref[...] = jnp.full_like(c_ref, -1.0)
    G = gt_ref.shape[1]

    def gbody(g, _):
        gy1 = gt_ref[0, g, 0]
        gx1 = gt_ref[0, g, 1]
        gy2 = gt_ref[0, g, 2]
        gx2 = gt_ref[0, g, 3]
        gid = ids_ref[0, g, 0]
        valid = ((jnp.abs(gy1) > 0) | (jnp.abs(gx1) > 0) |
                 (jnp.abs(gy2) > 0) | (jnp.abs(gx2) > 0))

        @pl.when(valid & (gid != 0))
        def _():
            a2 = (gy2 - gy1) * (gx2 - gx1)
            iy1 = jnp.maximum(y1, gy1)
            ix1 = jnp.maximum(x1, gx1)
            iy2 = jnp.minimum(y2, gy2)
            ix2 = jnp.minimum(x2, gx2)
            inter = jnp.maximum(iy2 - iy1, 0.0) * jnp.maximum(ix2 - ix1, 0.0)
            union = a1 + a2 - inter
            iou = inter / jnp.maximum(union, 1e-8)

            @pl.when(gid > 0)
            def _():
                nc_ref[...] = jnp.maximum(nc_ref[...], iou)

            @pl.when(gid < 0)
            def _():
                c_ref[...] = jnp.maximum(c_ref[...], iou)

        return ()

    jax.lax.fori_loop(0, G, gbody, ())

    roi_valid = ((jnp.abs(y1) > 0) | (jnp.abs(x1) > 0) |
                 (jnp.abs(y2) > 0) | (jnp.abs(x2) > 0))
    neg_one = jnp.float32(-1.0)
    nc_max = jnp.where(roi_valid, nc_ref[...], neg_one)
    c_max = jnp.where(roi_valid, c_ref[...], neg_one)
    pos = ((nc_max >= 0.5) & roi_valid).astype(jnp.float32)
    neg = ((nc_max < 0.5) & (c_max < 0.001) & roi_valid).astype(jnp.float32)
    out_ref[0, 0] = nc_max
    out_ref[0, 1] = c_max
    out_ref[0, 2] = pos
    out_ref[0, 3] = neg


def kernel(rois, gt_ids, gt_boxes):
    B, N, _ = rois.shape
    G = gt_boxes.shape[1]
    NS = 8
    NL = N // NS
    rois_t = jnp.transpose(rois, (0, 2, 1)).reshape(B, 4, NS, NL)
    out = pl.pallas_call(
        _detection_kernel,
        grid=(B,),
        in_specs=[
            pl.BlockSpec((1, 4, NS, NL), lambda b: (b, 0, 0, 0)),
            pl.BlockSpec((1, G, 1), lambda b: (b, 0, 0),
                         memory_space=pltpu.SMEM),
            pl.BlockSpec((1, G, 4), lambda b: (b, 0, 0),
                         memory_space=pltpu.SMEM),
        ],
        out_specs=pl.BlockSpec((1, 4, NS, NL), lambda b: (b, 0, 0, 0)),
        out_shape=jax.ShapeDtypeStruct((B, 4, NS, NL), jnp.float32),
        scratch_shapes=[
            pltpu.VMEM((NS, NL), jnp.float32),
            pltpu.VMEM((NS, NL), jnp.float32),
        ],
        compiler_params=pltpu.CompilerParams(
            dimension_semantics=("parallel",),
        ),
    )(rois_t, gt_ids.reshape(B, G, 1), gt_boxes)
    return out.reshape(B, 4, N)
